# baseline (speedup 1.0000x reference)
import functools

import jax
import jax.numpy as jnp
from jax import lax
from jax.experimental import pallas as pl
from jax.experimental.pallas import tpu as pltpu
from jax.experimental.pallas import tpu_sc as plsc

NG = 512
NA = 20000
NC = 2
NS = 16
NW = NC * NS
APW = 640
NAP = NW * APW
LASTW = NA - (NW - 1) * APW
NT = NG // 16
NCH = APW // 16
BG = 0.4
FG = 0.5

_MESH = plsc.VectorSubcoreMesh(core_axis_name="c", subcore_axis_name="s")


def _bcast(v, j):
    idx = jnp.full((16,), j, jnp.int32)
    return v.at[idx].get(mode="promise_in_bounds")


@functools.partial(
    pl.kernel,
    out_type=[
        jax.ShapeDtypeStruct((NG, NA), jnp.float32),
        jax.ShapeDtypeStruct((NAP,), jnp.float32),
        jax.ShapeDtypeStruct((NAP,), jnp.int32),
        jax.ShapeDtypeStruct((NW * NG,), jnp.float32),
    ],
    mesh=_MESH,
    compiler_params=pltpu.CompilerParams(use_tc_tiling_on_sc=False, needs_layout_passes=False),
    scratch_types=[
        pltpu.VMEM((NG,), jnp.float32),
        pltpu.VMEM((NG,), jnp.float32),
        pltpu.VMEM((NG,), jnp.float32),
        pltpu.VMEM((NG,), jnp.float32),
        pltpu.VMEM((NG,), jnp.float32),
        pltpu.VMEM((APW,), jnp.float32),
        pltpu.VMEM((APW,), jnp.float32),
        pltpu.VMEM((APW,), jnp.float32),
        pltpu.VMEM((APW,), jnp.float32),
        pltpu.VMEM((APW,), jnp.float32),
        pltpu.VMEM((APW,), jnp.float32),
        pltpu.VMEM((APW,), jnp.int32),
        pltpu.VMEM((16, APW), jnp.float32),
        pltpu.VMEM((NG,), jnp.float32),
    ],
)
def _pass_a(gx1_h, gy1_h, gx2_h, gy2_h, ax1_h, ay1_h, ax2_h, ay2_h,
            mat_h, cmax_h, cidx_h, rmax_h,
            gx1, gy1, gx2, gy2, ga, ax1, ay1, ax2, ay2, aa,
            cmv, civ, buf, rmw):
    wid = lax.axis_index("s") * NC + lax.axis_index("c")
    base = wid * APW

    pltpu.sync_copy(gx1_h, gx1)
    pltpu.sync_copy(gy1_h, gy1)
    pltpu.sync_copy(gx2_h, gx2)
    pltpu.sync_copy(gy2_h, gy2)
    pltpu.sync_copy(ax1_h.at[pl.ds(base, APW)], ax1)
    pltpu.sync_copy(ay1_h.at[pl.ds(base, APW)], ay1)
    pltpu.sync_copy(ax2_h.at[pl.ds(base, APW)], ax2)
    pltpu.sync_copy(ay2_h.at[pl.ds(base, APW)], ay2)

    def _garea(t, _):
        s = pl.ds(t * 16, 16)
        ga[s] = (gx2[s] - gx1[s]) * (gy2[s] - gy1[s])
        return 0
    lax.fori_loop(0, NT, _garea, 0)

    def _aarea(c, _):
        s = pl.ds(c * 16, 16)
        aa[s] = (ax2[s] - ax1[s]) * (ay2[s] - ay1[s])
        cmv[s] = jnp.full((16,), -1.0, jnp.float32)
        civ[s] = jnp.zeros((16,), jnp.int32)
        return 0
    lax.fori_loop(0, NCH, _aarea, 0)

    lane = jnp.arange(16, dtype=jnp.int32)
    zeros16 = jnp.zeros((16,), jnp.float32)

    def _gt_group(t, _):
        gs = pl.ds(t * 16, 16)
        g1 = gx1[gs]
        g2 = gy1[gs]
        g3 = gx2[gs]
        g4 = gy2[gs]
        gar = ga[gs]

        def _chunk(c, racc):
            s = pl.ds(c * 16, 16)
            a1 = ax1[s]
            a2 = ay1[s]
            a3 = ax2[s]
            a4 = ay2[s]
            aar = aa[s]
            cm = cmv[s]
            ci = civ[s]
            racc = list(racc)
            for j in range(16):
                g1b = _bcast(g1, j)
                g2b = _bcast(g2, j)
                g3b = _bcast(g3, j)
                g4b = _bcast(g4, j)
                gab = _bcast(gar, j)
                ltx = jnp.maximum(a1, g1b)
                lty = jnp.maximum(a2, g2b)
                rbx = jnp.minimum(a3, g3b)
                rby = jnp.minimum(a4, g4b)
                w = jnp.maximum(rbx - ltx, 0.0)
                h = jnp.maximum(rby - lty, 0.0)
                inter = w * h
                union = (gab + aar) - inter
                iou = inter / union
                buf[j, s] = iou
                racc[j] = jnp.maximum(racc[j], iou)
                gidx = t * 16 + j
                ci = jnp.where(iou > cm, gidx, ci)
                cm = jnp.maximum(cm, iou)
            cmv[s] = cm
            civ[s] = ci
            return tuple(racc)

        racc = lax.fori_loop(0, NCH, _chunk, tuple([zeros16] * 16))

        @pl.when(wid != NW - 1)
        def _():
            pltpu.sync_copy(buf, mat_h.at[pl.ds(t * 16, 16), pl.ds(base, APW)])

        @pl.when(wid == NW - 1)
        def _():
            pltpu.sync_copy(buf.at[:, pl.ds(0, LASTW)],
                            mat_h.at[pl.ds(t * 16, 16), pl.ds(NA - LASTW, LASTW)])

        acc = zeros16
        for j in range(16):
            m = jnp.max(racc[j])
            acc = jnp.where(lane == j, m, acc)
        rmw[gs] = acc
        return 0

    lax.fori_loop(0, NT, _gt_group, 0)

    pltpu.sync_copy(cmv, cmax_h.at[pl.ds(base, APW)])
    pltpu.sync_copy(civ, cidx_h.at[pl.ds(base, APW)])
    pltpu.sync_copy(rmw, rmax_h.at[pl.ds(wid * NG, NG)])


@functools.partial(
    pl.kernel,
    out_type=[jax.ShapeDtypeStruct((NAP,), jnp.int32)],
    mesh=_MESH,
    compiler_params=pltpu.CompilerParams(use_tc_tiling_on_sc=False, needs_layout_passes=False),
    scratch_types=[
        pltpu.VMEM((NW * NG,), jnp.float32),
        pltpu.VMEM((NG,), jnp.float32),
        pltpu.VMEM((16, APW), jnp.float32),
        pltpu.VMEM((APW,), jnp.float32),
        pltpu.VMEM((APW,), jnp.int32),
        pltpu.VMEM((APW,), jnp.int32),
        pltpu.VMEM((APW,), jnp.int32),
    ],
)
def _pass_b(mat_h, rmax_h, cmax_h, cidx_h, idx_h,
            rall, grm, mrow, cmv, civ, resv, outv):
    wid = lax.axis_index("s") * NC + lax.axis_index("c")
    base = wid * APW

    pltpu.sync_copy(rmax_h, rall)
    pltpu.sync_copy(cmax_h.at[pl.ds(base, APW)], cmv)
    pltpu.sync_copy(cidx_h.at[pl.ds(base, APW)], civ)

    def _reduce(t, _):
        acc = rall[pl.ds(t * 16, 16)]
        for w in range(1, NW):
            acc = jnp.maximum(acc, rall[pl.ds(w * NG + t * 16, 16)])
        grm[pl.ds(t * 16, 16)] = acc
        return 0
    lax.fori_loop(0, NT, _reduce, 0)

    def _zero(c, _):
        resv[pl.ds(c * 16, 16)] = jnp.zeros((16,), jnp.int32)
        return 0
    lax.fori_loop(0, NCH, _zero, 0)

    def _gt_group(t, _):
        @pl.when(wid != NW - 1)
        def _():
            pltpu.sync_copy(mat_h.at[pl.ds(t * 16, 16), pl.ds(base, APW)], mrow)

        @pl.when(wid == NW - 1)
        def _():
            pltpu.sync_copy(mat_h.at[pl.ds(t * 16, 16), pl.ds(NA - LASTW, LASTW)],
                            mrow.at[:, pl.ds(0, LASTW)])

        gv = grm[pl.ds(t * 16, 16)]
        for j in range(16):
            gb = _bcast(gv, j)

            def _chunk(c, _, j=j, gb=gb):
                s = pl.ds(c * 16, 16)
                v = mrow[j, s]
                resv[s] = jnp.where(v == gb, jnp.int32(1), resv[s])
                return 0
            lax.fori_loop(0, NCH, _chunk, 0)
        return 0
    lax.fori_loop(0, NT, _gt_group, 0)

    def _final(c, _):
        s = pl.ds(c * 16, 16)
        mx = cmv[s]
        ix = civ[s]
        m = jnp.where(mx < BG, jnp.int32(-1), ix)
        m = jnp.where((mx >= BG) & (mx < FG), jnp.int32(-2), m)
        m = jnp.where(resv[s] > 0, ix, m)
        outv[s] = m
        return 0
    lax.fori_loop(0, NCH, _final, 0)

    pltpu.sync_copy(outv, idx_h.at[pl.ds(base, APW)])


def kernel(gt_boxes, anchors):
    ap = jnp.concatenate(
        [anchors, jnp.zeros((NAP - NA, 4), jnp.float32)], axis=0)
    mat, cmax, cidx, rmax = _pass_a(
        gt_boxes[:, 0], gt_boxes[:, 1], gt_boxes[:, 2], gt_boxes[:, 3],
        ap[:, 0], ap[:, 1], ap[:, 2], ap[:, 3])
    idxp, = _pass_b(mat, rmax, cmax, cidx)
    return mat, idxp[:NA].astype(jnp.int64)


# speedup vs baseline: 1.1569x; 1.1569x over previous
import functools

import jax
import jax.numpy as jnp
from jax import lax
from jax.experimental import pallas as pl
from jax.experimental.pallas import tpu as pltpu
from jax.experimental.pallas import tpu_sc as plsc

NG = 512
NA = 20000
NC = 2
NS = 16
NW = NC * NS
APW = 640
NAP = NW * APW
LASTW = NA - (NW - 1) * APW
NT = NG // 16
NCH = APW // 16
BG = 0.4
FG = 0.5

_MESH = plsc.VectorSubcoreMesh(core_axis_name="c", subcore_axis_name="s")


def _bcast(v, j):
    idx = jnp.full((16,), j, jnp.int32)
    return v.at[idx].get(mode="promise_in_bounds")


@functools.partial(
    pl.kernel,
    out_type=[
        jax.ShapeDtypeStruct((NG, NA), jnp.float32),
        jax.ShapeDtypeStruct((NAP,), jnp.float32),
        jax.ShapeDtypeStruct((NAP,), jnp.int32),
        jax.ShapeDtypeStruct((NW * NG,), jnp.float32),
    ],
    mesh=_MESH,
    compiler_params=pltpu.CompilerParams(use_tc_tiling_on_sc=False, needs_layout_passes=False),
    scratch_types=[
        pltpu.VMEM((NG,), jnp.float32),
        pltpu.VMEM((NG,), jnp.float32),
        pltpu.VMEM((NG,), jnp.float32),
        pltpu.VMEM((NG,), jnp.float32),
        pltpu.VMEM((NG,), jnp.float32),
        pltpu.VMEM((APW,), jnp.float32),
        pltpu.VMEM((APW,), jnp.float32),
        pltpu.VMEM((APW,), jnp.float32),
        pltpu.VMEM((APW,), jnp.float32),
        pltpu.VMEM((APW,), jnp.float32),
        pltpu.VMEM((APW,), jnp.float32),
        pltpu.VMEM((APW,), jnp.int32),
        pltpu.VMEM((16, APW), jnp.float32),
        pltpu.VMEM((NG,), jnp.float32),
    ],
)
def _pass_a(gx1_h, gy1_h, gx2_h, gy2_h, ax1_h, ay1_h, ax2_h, ay2_h,
            mat_h, cmax_h, cidx_h, rmax_h,
            gx1, gy1, gx2, gy2, ga, ax1, ay1, ax2, ay2, aa,
            cmv, civ, buf, rmw):
    wid = lax.axis_index("s") * NC + lax.axis_index("c")
    base = wid * APW

    pltpu.sync_copy(gx1_h, gx1)
    pltpu.sync_copy(gy1_h, gy1)
    pltpu.sync_copy(gx2_h, gx2)
    pltpu.sync_copy(gy2_h, gy2)
    pltpu.sync_copy(ax1_h.at[pl.ds(base, APW)], ax1)
    pltpu.sync_copy(ay1_h.at[pl.ds(base, APW)], ay1)
    pltpu.sync_copy(ax2_h.at[pl.ds(base, APW)], ax2)
    pltpu.sync_copy(ay2_h.at[pl.ds(base, APW)], ay2)

    def _garea(t, _):
        s = pl.ds(t * 16, 16)
        ga[s] = (gx2[s] - gx1[s]) * (gy2[s] - gy1[s])
        return 0
    lax.fori_loop(0, NT, _garea, 0)

    def _aarea(c, _):
        s = pl.ds(c * 16, 16)
        aa[s] = (ax2[s] - ax1[s]) * (ay2[s] - ay1[s])
        cmv[s] = jnp.full((16,), -1.0, jnp.float32)
        civ[s] = jnp.zeros((16,), jnp.int32)
        return 0
    lax.fori_loop(0, NCH, _aarea, 0)

    lane = jnp.arange(16, dtype=jnp.int32)
    zeros16 = jnp.zeros((16,), jnp.float32)

    def _gt_group(t, _):
        gs = pl.ds(t * 16, 16)
        g1 = gx1[gs]
        g2 = gy1[gs]
        g3 = gx2[gs]
        g4 = gy2[gs]
        gar = ga[gs]

        def _chunk(c, racc):
            s = pl.ds(c * 16, 16)
            a1 = ax1[s]
            a2 = ay1[s]
            a3 = ax2[s]
            a4 = ay2[s]
            aar = aa[s]
            cm = cmv[s]
            ci = civ[s]
            racc = list(racc)
            for j in range(16):
                g1b = _bcast(g1, j)
                g2b = _bcast(g2, j)
                g3b = _bcast(g3, j)
                g4b = _bcast(g4, j)
                gab = _bcast(gar, j)
                ltx = jnp.maximum(a1, g1b)
                lty = jnp.maximum(a2, g2b)
                rbx = jnp.minimum(a3, g3b)
                rby = jnp.minimum(a4, g4b)
                w = jnp.maximum(rbx - ltx, 0.0)
                h = jnp.maximum(rby - lty, 0.0)
                inter = w * h
                union = (gab + aar) - inter
                iou = inter / union
                buf[j, s] = iou
                racc[j] = jnp.maximum(racc[j], iou)
                gidx = t * 16 + j
                ci = jnp.where(iou > cm, gidx, ci)
                cm = jnp.maximum(cm, iou)
            cmv[s] = cm
            civ[s] = ci
            return tuple(racc)

        racc = plsc.parallel_loop(0, NCH, unroll=2,
                                  carry=tuple([zeros16] * 16))(
            lambda c, r: _chunk(c, r))

        @pl.when(wid != NW - 1)
        def _():
            pltpu.sync_copy(buf, mat_h.at[pl.ds(t * 16, 16), pl.ds(base, APW)])

        @pl.when(wid == NW - 1)
        def _():
            pltpu.sync_copy(buf.at[:, pl.ds(0, LASTW)],
                            mat_h.at[pl.ds(t * 16, 16), pl.ds(NA - LASTW, LASTW)])

        acc = zeros16
        for j in range(16):
            m = jnp.max(racc[j])
            acc = jnp.where(lane == j, m, acc)
        rmw[gs] = acc
        return 0

    lax.fori_loop(0, NT, _gt_group, 0)

    pltpu.sync_copy(cmv, cmax_h.at[pl.ds(base, APW)])
    pltpu.sync_copy(civ, cidx_h.at[pl.ds(base, APW)])
    pltpu.sync_copy(rmw, rmax_h.at[pl.ds(wid * NG, NG)])


@functools.partial(
    pl.kernel,
    out_type=[jax.ShapeDtypeStruct((NAP,), jnp.int32)],
    mesh=_MESH,
    compiler_params=pltpu.CompilerParams(use_tc_tiling_on_sc=False, needs_layout_passes=False),
    scratch_types=[
        pltpu.VMEM((NW * NG,), jnp.float32),
        pltpu.VMEM((NG,), jnp.float32),
        pltpu.VMEM((16, APW), jnp.float32),
        pltpu.VMEM((APW,), jnp.float32),
        pltpu.VMEM((APW,), jnp.int32),
        pltpu.VMEM((APW,), jnp.int32),
        pltpu.VMEM((APW,), jnp.int32),
    ],
)
def _pass_b(mat_h, rmax_h, cmax_h, cidx_h, idx_h,
            rall, grm, mrow, cmv, civ, resv, outv):
    wid = lax.axis_index("s") * NC + lax.axis_index("c")
    base = wid * APW

    pltpu.sync_copy(rmax_h, rall)
    pltpu.sync_copy(cmax_h.at[pl.ds(base, APW)], cmv)
    pltpu.sync_copy(cidx_h.at[pl.ds(base, APW)], civ)

    def _reduce(t, _):
        acc = rall[pl.ds(t * 16, 16)]
        for w in range(1, NW):
            acc = jnp.maximum(acc, rall[pl.ds(w * NG + t * 16, 16)])
        grm[pl.ds(t * 16, 16)] = acc
        return 0
    lax.fori_loop(0, NT, _reduce, 0)

    def _zero(c, _):
        resv[pl.ds(c * 16, 16)] = jnp.zeros((16,), jnp.int32)
        return 0
    lax.fori_loop(0, NCH, _zero, 0)

    def _gt_group(t, _):
        @pl.when(wid != NW - 1)
        def _():
            pltpu.sync_copy(mat_h.at[pl.ds(t * 16, 16), pl.ds(base, APW)], mrow)

        @pl.when(wid == NW - 1)
        def _():
            pltpu.sync_copy(mat_h.at[pl.ds(t * 16, 16), pl.ds(NA - LASTW, LASTW)],
                            mrow.at[:, pl.ds(0, LASTW)])

        gv = grm[pl.ds(t * 16, 16)]
        gbs = [_bcast(gv, j) for j in range(16)]

        def _cchunk(c):
            s = pl.ds(c * 16, 16)
            r = resv[s]
            for j in range(16):
                r = jnp.where(mrow[j, s] == gbs[j], jnp.int32(1), r)
            resv[s] = r
        plsc.parallel_loop(0, NCH, unroll=2)(_cchunk)
        return 0
    lax.fori_loop(0, NT, _gt_group, 0)

    def _final(c, _):
        s = pl.ds(c * 16, 16)
        mx = cmv[s]
        ix = civ[s]
        m = jnp.where(mx < BG, jnp.int32(-1), ix)
        m = jnp.where((mx >= BG) & (mx < FG), jnp.int32(-2), m)
        m = jnp.where(resv[s] > 0, ix, m)
        outv[s] = m
        return 0
    lax.fori_loop(0, NCH, _final, 0)

    pltpu.sync_copy(outv, idx_h.at[pl.ds(base, APW)])


def kernel(gt_boxes, anchors):
    ap = jnp.concatenate(
        [anchors, jnp.zeros((NAP - NA, 4), jnp.float32)], axis=0)
    mat, cmax, cidx, rmax = _pass_a(
        gt_boxes[:, 0], gt_boxes[:, 1], gt_boxes[:, 2], gt_boxes[:, 3],
        ap[:, 0], ap[:, 1], ap[:, 2], ap[:, 3])
    idxp, = _pass_b(mat, rmax, cmax, cidx)
    return mat, idxp[:NA].astype(jnp.int64)
